# Initial kernel scaffold; baseline (speedup 1.0000x reference)
import jax
import jax.numpy as jnp
from jax.experimental import pallas as pl


def kernel(adjs, feature, idxs, adjs_hidden, features_hidden):
    raise NotImplementedError("write your pallas kernel here")


# speedup vs baseline: 2.3829x; 2.3829x over previous
import functools

import jax
import jax.numpy as jnp
from jax import lax
from jax.experimental import pallas as pl
from jax.experimental.pallas import tpu as pltpu
from jax.experimental.pallas import tpu_sc as plsc

_N_FILTER = 8
_S_SUB = 5
_D_IN = 128
_K_STEP = 3
_TAO = 0.05
_N_NODES = 100000
_N_SUB = 50000

_BS = 400
_GRID = _N_SUB // _BS

_CHUNK = 200
_N_FLAT = _S_SUB * _N_SUB
_N_CHUNKS = _N_FLAT // _CHUNK
_NW = 32

_PAIR = (
    (None, 0, 1, 2, 3),
    (0, None, 4, 5, 6),
    (1, 4, None, 7, 8),
    (2, 5, 7, None, 9),
    (3, 6, 8, 9, None),
)

def _rb(x):
    return x.astype(jnp.bfloat16).astype(jnp.float32)


def _sc_gather(table, idx_flat):
    mesh = plsc.VectorSubcoreMesh(core_axis_name="c", subcore_axis_name="s")

    @functools.partial(
        pl.kernel,
        mesh=mesh,
        out_type=jax.ShapeDtypeStruct((_N_FLAT, _D_IN), jnp.float32),
        scratch_types=[
            pltpu.VMEM((_CHUNK,), jnp.int32),
            pltpu.VMEM((_CHUNK, _D_IN), jnp.float32),
            pltpu.SemaphoreType.DMA,
        ],
    )
    def gk(table_hbm, idx_hbm, out_hbm, idx_v, rows_v, sem):
        wid = lax.axis_index("s") * 2 + lax.axis_index("c")
        nloc = (_N_CHUNKS - wid + _NW - 1) // _NW

        def body(i, carry):
            base = (wid + i * _NW) * _CHUNK
            pltpu.sync_copy(idx_hbm.at[pl.ds(base, _CHUNK)], idx_v)
            pltpu.async_copy(table_hbm.at[idx_v], rows_v, sem).wait()
            pltpu.sync_copy(rows_v, out_hbm.at[pl.ds(base, _CHUNK)])
            return carry

        lax.fori_loop(0, nloc, body, 0)

    return gk(table, idx_flat)


def _tc_body(g_ref, adj_ref, val_ref, ah_ref, fh_ref, out_ref, fhh_scr, fhsq_scr):
    @pl.when(pl.program_id(0) == 0)
    def _():
        sig = 1.0 / (1.0 + jnp.exp(-ah_ref[...]))
        sig = _rb(sig)
        for d in range(_S_SUB):
            for b in range(_N_FILTER):
                r = d * _N_FILTER + b
                fhh_scr[0, r:r + 1, :] = fh_ref[b, d:d + 1, :]
        for h in range(1, _K_STEP):
            for d in range(_S_SUB):
                acc = None
                for c in range(_S_SUB):
                    if c == d:
                        continue
                    k = _PAIR[d][c]
                    term = sig[:, k:k + 1] * _rb(fhh_scr[h - 1, c * 8:(c + 1) * 8, :])
                    acc = term if acc is None else acc + term
                fhh_scr[h, d * 8:(d + 1) * 8, :] = acc
        ones_row = jnp.ones((1, _D_IN), jnp.bfloat16)
        for h in range(_K_STEP):
            w = fhh_scr[h]
            fhsq_scr[h, :, :] = lax.dot_general(
                ones_row, (w * w).astype(jnp.bfloat16), (((1,), (1,)), ((), ())),
                preferred_element_type=jnp.float32)

    F = [g_ref[c] * val_ref[:, c:c + 1] for c in range(_S_SUB)]

    ones_col = jnp.ones((_D_IN, 1), jnp.bfloat16)
    T = [None] * _S_SUB
    for h in range(_K_STEP):
        if h > 0:
            adjb = _rb(adj_ref[...])
            Fb = [_rb(f) for f in F]
            newF = []
            for r in range(_S_SUB):
                acc = adjb[:, r * 5:r * 5 + 1] * Fb[0]
                for c in range(1, _S_SUB):
                    acc = acc + adjb[:, r * 5 + c:r * 5 + c + 1] * Fb[c]
                newF.append(acc)
            F = newF
        w16 = fhh_scr[h].astype(jnp.bfloat16)
        q = fhsq_scr[h]
        for c in range(_S_SUB):
            M = lax.dot_general(F[c].astype(jnp.bfloat16), w16,
                                (((1,), (1,)), ((), ())),
                                preferred_element_type=jnp.float32)
            fsq = lax.dot_general((F[c] * F[c]).astype(jnp.bfloat16), ones_col,
                                  (((1,), (0,)), ((), ())),
                                  preferred_element_type=jnp.float32)
            e = jnp.exp(-(fsq + q - 2.0 * M) / _D_IN / _TAO)
            T[c] = e if h == 0 else T[c] + e

    out = T[0][:, 0:_N_FILTER]
    neg = jnp.float32(-1.0)
    blocked = [jnp.full((_BS, _N_FILTER), d == 0, jnp.bool_) for d in range(_S_SUB)]
    for i in range(1, _S_SUB):
        v = [jnp.where(blocked[d], neg, T[i][:, d * 8:(d + 1) * 8])
             for d in range(_S_SUB)]
        m = v[0]
        for d in range(1, _S_SUB):
            m = jnp.maximum(m, v[d])
        out = out + m
        found = jnp.zeros((_BS, _N_FILTER), jnp.bool_)
        for d in range(_S_SUB):
            hit = (v[d] == m) & jnp.logical_not(found)
            blocked[d] = blocked[d] | hit
            found = found | hit
    out_ref[...] = out


def _tc_compute(g3, adjs2d, valid, adjs_hidden, features_hidden, interpret=False):
    return pl.pallas_call(
        _tc_body,
        grid=(_GRID,),
        in_specs=[
            pl.BlockSpec((_S_SUB, _BS, _D_IN), lambda i: (0, i, 0)),
            pl.BlockSpec((_BS, _S_SUB * _S_SUB), lambda i: (i, 0)),
            pl.BlockSpec((_BS, _S_SUB), lambda i: (i, 0)),
            pl.BlockSpec((_N_FILTER, 10), lambda i: (0, 0)),
            pl.BlockSpec((_N_FILTER, _S_SUB, _D_IN), lambda i: (0, 0, 0)),
        ],
        out_specs=pl.BlockSpec((_BS, _N_FILTER), lambda i: (i, 0)),
        out_shape=jax.ShapeDtypeStruct((_N_SUB, _N_FILTER), jnp.float32),
        scratch_shapes=[
            pltpu.VMEM((_K_STEP, 40, _D_IN), jnp.float32),
            pltpu.VMEM((_K_STEP, 1, 40), jnp.float32),
        ],
        compiler_params=pltpu.CompilerParams(
            dimension_semantics=("arbitrary",)),
        interpret=interpret,
    )(g3, adjs2d, valid, adjs_hidden, features_hidden)


def kernel(adjs, feature, idxs, adjs_hidden, features_hidden):
    idx32 = idxs.astype(jnp.int32)
    valid = (idx32 < _N_NODES).astype(jnp.float32)
    idx_cm = jnp.minimum(idx32, _N_NODES - 1).T.reshape(-1)
    g_flat = _sc_gather(feature, idx_cm)
    g3 = g_flat.reshape(_S_SUB, _N_SUB, _D_IN)
    adjs2d = adjs.reshape(_N_SUB, _S_SUB * _S_SUB)
    return _tc_compute(g3, adjs2d, valid, adjs_hidden, features_hidden)


# speedup vs baseline: 4.1172x; 1.7278x over previous
import functools

import jax
import jax.numpy as jnp
from jax import lax
from jax.experimental import pallas as pl
from jax.experimental.pallas import tpu as pltpu
from jax.experimental.pallas import tpu_sc as plsc

_N_FILTER = 8
_S_SUB = 5
_D_IN = 128
_K_STEP = 3
_TAO = 0.05
_N_NODES = 100000
_N_SUB = 50000

_N_PAD = 50176
_BS = 512
_GRID = _N_PAD // _BS

_CHUNK = 256
_N_FLAT = _S_SUB * _N_PAD
_N_CHUNKS = _N_FLAT // _CHUNK
_NW = 32

_PAIR = (
    (None, 0, 1, 2, 3),
    (0, None, 4, 5, 6),
    (1, 4, None, 7, 8),
    (2, 5, 7, None, 9),
    (3, 6, 8, 9, None),
)

def _rb(x):
    return x.astype(jnp.bfloat16).astype(jnp.float32)


def _sc_gather(table, idx_flat):
    mesh = plsc.VectorSubcoreMesh(core_axis_name="c", subcore_axis_name="s")

    @functools.partial(
        pl.kernel,
        mesh=mesh,
        out_type=jax.ShapeDtypeStruct((_N_FLAT, _D_IN), jnp.float32),
        scratch_types=[
            pltpu.VMEM((_CHUNK,), jnp.int32),
            pltpu.VMEM((_CHUNK, _D_IN), jnp.float32),
            pltpu.SemaphoreType.DMA,
        ],
    )
    def gk(table_hbm, idx_hbm, out_hbm, idx_v, rows_v, sem):
        wid = lax.axis_index("s") * 2 + lax.axis_index("c")
        nloc = (_N_CHUNKS - wid + _NW - 1) // _NW

        def body(i, carry):
            base = (wid + i * _NW) * _CHUNK
            pltpu.sync_copy(idx_hbm.at[pl.ds(base, _CHUNK)], idx_v)
            pltpu.async_copy(table_hbm.at[idx_v], rows_v, sem).wait()
            pltpu.sync_copy(rows_v, out_hbm.at[pl.ds(base, _CHUNK)])
            return carry

        lax.fori_loop(0, nloc, body, 0)

    return gk(table, idx_flat)


def _tc_body(g_ref, adj_ref, val_ref, ah_ref, fh_ref, out_ref, fhh_scr, fhsq_scr):
    @pl.when(pl.program_id(0) == 0)
    def _():
        sig = 1.0 / (1.0 + jnp.exp(-ah_ref[...]))
        sig = _rb(sig)
        for d in range(_S_SUB):
            for b in range(_N_FILTER):
                r = d * _N_FILTER + b
                fhh_scr[0, r:r + 1, :] = fh_ref[b, d:d + 1, :]
        for h in range(1, _K_STEP):
            for d in range(_S_SUB):
                acc = None
                for c in range(_S_SUB):
                    if c == d:
                        continue
                    k = _PAIR[d][c]
                    term = sig[:, k:k + 1] * _rb(fhh_scr[h - 1, c * 8:(c + 1) * 8, :])
                    acc = term if acc is None else acc + term
                fhh_scr[h, d * 8:(d + 1) * 8, :] = acc
        ones_col = jnp.ones((_D_IN, 1), jnp.bfloat16)
        for h in range(_K_STEP):
            w = fhh_scr[h]
            fhsq_scr[h, :, :] = lax.dot_general(
                (w * w).astype(jnp.bfloat16), ones_col, (((1,), (0,)), ((), ())),
                preferred_element_type=jnp.float32)

    F = [g_ref[c] * val_ref[:, c:c + 1] for c in range(_S_SUB)]

    ones_row16 = jnp.ones((1, _D_IN), jnp.bfloat16)
    adjb = _rb(adj_ref[...])
    T = [None] * _S_SUB
    for h in range(_K_STEP):
        if h > 0:
            Fb = [_rb(f) for f in F]
            newF = []
            for r in range(_S_SUB):
                acc = adjb[:, r * 5:r * 5 + 1] * Fb[0]
                for c in range(1, _S_SUB):
                    acc = acc + adjb[:, r * 5 + c:r * 5 + c + 1] * Fb[c]
                newF.append(acc)
            F = newF
        w16 = fhh_scr[h].astype(jnp.bfloat16)
        q = fhsq_scr[h]
        for c in range(_S_SUB):
            Mt = lax.dot_general(w16, F[c].astype(jnp.bfloat16),
                                 (((1,), (1,)), ((), ())),
                                 preferred_element_type=jnp.float32)
            fsqt = lax.dot_general(ones_row16, (F[c] * F[c]).astype(jnp.bfloat16),
                                   (((1,), (1,)), ((), ())),
                                   preferred_element_type=jnp.float32)
            e = jnp.exp(-(fsqt + q - 2.0 * Mt) / _D_IN / _TAO)
            T[c] = e if h == 0 else T[c] + e

    out = T[0][0:_N_FILTER, :]
    neg = jnp.float32(-1.0)
    blocked = [jnp.full((_N_FILTER, _BS), d == 0, jnp.bool_) for d in range(_S_SUB)]
    for i in range(1, _S_SUB):
        v = [jnp.where(blocked[d], neg, T[i][d * 8:(d + 1) * 8, :])
             for d in range(_S_SUB)]
        m = v[0]
        for d in range(1, _S_SUB):
            m = jnp.maximum(m, v[d])
        out = out + m
        found = jnp.zeros((_N_FILTER, _BS), jnp.bool_)
        for d in range(_S_SUB):
            hit = (v[d] == m) & jnp.logical_not(found)
            blocked[d] = blocked[d] | hit
            found = found | hit
    out_ref[...] = out


def _tc_compute(g3, adjs2d, valid, adjs_hidden, features_hidden, interpret=False):
    return pl.pallas_call(
        _tc_body,
        grid=(_GRID,),
        in_specs=[
            pl.BlockSpec((_S_SUB, _BS, _D_IN), lambda i: (0, i, 0)),
            pl.BlockSpec((_BS, _S_SUB * _S_SUB), lambda i: (i, 0)),
            pl.BlockSpec((_BS, _S_SUB), lambda i: (i, 0)),
            pl.BlockSpec((_N_FILTER, 10), lambda i: (0, 0)),
            pl.BlockSpec((_N_FILTER, _S_SUB, _D_IN), lambda i: (0, 0, 0)),
        ],
        out_specs=pl.BlockSpec((_N_FILTER, _BS), lambda i: (0, i)),
        out_shape=jax.ShapeDtypeStruct((_N_FILTER, _N_PAD), jnp.float32),
        scratch_shapes=[
            pltpu.VMEM((_K_STEP, 40, _D_IN), jnp.float32),
            pltpu.VMEM((_K_STEP, 40, 1), jnp.float32),
        ],
        compiler_params=pltpu.CompilerParams(
            dimension_semantics=("arbitrary",)),
        interpret=interpret,
    )(g3, adjs2d, valid, adjs_hidden, features_hidden)


def kernel(adjs, feature, idxs, adjs_hidden, features_hidden):
    pad = _N_PAD - _N_SUB
    idx32 = idxs.astype(jnp.int32)
    valid = jnp.pad((idx32 < _N_NODES).astype(jnp.float32),
                    ((0, pad), (0, 0)))
    idx_cm = jnp.pad(jnp.minimum(idx32, _N_NODES - 1).T,
                     ((0, 0), (0, pad))).reshape(-1)
    g_flat = _sc_gather(feature, idx_cm)
    g3 = g_flat.reshape(_S_SUB, _N_PAD, _D_IN)
    adjs2d = jnp.pad(adjs.reshape(_N_SUB, _S_SUB * _S_SUB),
                     ((0, pad), (0, 0)))
    out_t = _tc_compute(g3, adjs2d, valid, adjs_hidden, features_hidden)
    return out_t[:, :_N_SUB].T


# speedup vs baseline: 4.1656x; 1.0118x over previous
import functools

import jax
import jax.numpy as jnp
from jax import lax
from jax.experimental import pallas as pl
from jax.experimental.pallas import tpu as pltpu
from jax.experimental.pallas import tpu_sc as plsc

_N_FILTER = 8
_S_SUB = 5
_D_IN = 128
_K_STEP = 3
_TAO = 0.05
_N_NODES = 100000
_N_SUB = 50000

_N_PAD = 50176
_BS = 512
_GRID = _N_PAD // _BS

_NW = 32
_N_FLAT = _S_SUB * _N_PAD
_ROWS_W = _N_FLAT // _NW
_CHUNK = 392
_NCH_W = _ROWS_W // _CHUNK

_PAIR = (
    (None, 0, 1, 2, 3),
    (0, None, 4, 5, 6),
    (1, 4, None, 7, 8),
    (2, 5, 7, None, 9),
    (3, 6, 8, 9, None),
)

def _rb(x):
    return x.astype(jnp.bfloat16).astype(jnp.float32)


def _sc_gather(table, idx_flat):
    mesh = plsc.VectorSubcoreMesh(core_axis_name="c", subcore_axis_name="s")

    @functools.partial(
        pl.kernel,
        mesh=mesh,
        out_type=jax.ShapeDtypeStruct((_N_FLAT, _D_IN), jnp.float32),
        scratch_types=[
            pltpu.VMEM((_CHUNK,), jnp.int32),
            pltpu.VMEM((_CHUNK,), jnp.int32),
            pltpu.VMEM((_CHUNK, _D_IN), jnp.float32),
            pltpu.VMEM((_CHUNK, _D_IN), jnp.float32),
            pltpu.SemaphoreType.DMA,
            pltpu.SemaphoreType.DMA,
            pltpu.SemaphoreType.DMA,
            pltpu.SemaphoreType.DMA,
        ],
    )
    def gk(table_hbm, idx_hbm, out_hbm, ixa, ixb, rwa, rwb, g0, g1, o0, o1):
        wid = lax.axis_index("s") * 2 + lax.axis_index("c")
        base = wid * _ROWS_W
        bufs = ((ixa, rwa, g0, o0), (ixb, rwb, g1, o1))

        for b in range(2):
            ix, rw, g, _o = bufs[b]
            pltpu.sync_copy(idx_hbm.at[pl.ds(base + b * _CHUNK, _CHUNK)], ix)
            pltpu.async_copy(table_hbm.at[ix], rw, g)

        def body(j, carry):
            for b in range(2):
                ix, rw, g, o = bufs[b]
                off = base + (2 * j + b) * _CHUNK
                pltpu.make_async_copy(table_hbm.at[ix], rw, g).wait()
                pltpu.async_copy(rw, out_hbm.at[pl.ds(off, _CHUNK)], o)

                @pl.when(j < _NCH_W // 2 - 1)
                def _():
                    pltpu.make_async_copy(
                        rw, out_hbm.at[pl.ds(off, _CHUNK)], o).wait()
                    pltpu.sync_copy(
                        idx_hbm.at[pl.ds(off + 2 * _CHUNK, _CHUNK)], ix)
                    pltpu.async_copy(table_hbm.at[ix], rw, g)
            return carry

        lax.fori_loop(0, _NCH_W // 2, body, 0)
        for b in range(2):
            ix, rw, g, o = bufs[b]
            pltpu.make_async_copy(
                rw,
                out_hbm.at[pl.ds(base + (_NCH_W - 2 + b) * _CHUNK, _CHUNK)],
                o).wait()

    return gk(table, idx_flat)


def _tc_body(g_ref, adj_ref, val_ref, ah_ref, fh_ref, out_ref, fhh_scr,
             fhh16_scr, fhsq_scr):
    @pl.when(pl.program_id(0) == 0)
    def _():
        sig = 1.0 / (1.0 + jnp.exp(-ah_ref[...]))
        sig = _rb(sig)
        for d in range(_S_SUB):
            for b in range(_N_FILTER):
                r = d * _N_FILTER + b
                fhh_scr[0, r:r + 1, :] = fh_ref[b, d:d + 1, :]
        for h in range(1, _K_STEP):
            for d in range(_S_SUB):
                acc = None
                for c in range(_S_SUB):
                    if c == d:
                        continue
                    k = _PAIR[d][c]
                    term = sig[:, k:k + 1] * _rb(fhh_scr[h - 1, c * 8:(c + 1) * 8, :])
                    acc = term if acc is None else acc + term
                fhh_scr[h, d * 8:(d + 1) * 8, :] = acc
        ones_col = jnp.ones((_D_IN, 1), jnp.bfloat16)
        for h in range(_K_STEP):
            w = fhh_scr[h]
            fhh16_scr[h, :, :] = w.astype(jnp.bfloat16)
            fhsq_scr[h, :, :] = lax.dot_general(
                (w * w).astype(jnp.bfloat16), ones_col, (((1,), (0,)), ((), ())),
                preferred_element_type=jnp.float32)

    F = [g_ref[c] * val_ref[:, c:c + 1] for c in range(_S_SUB)]

    ones_row16 = jnp.ones((1, _D_IN), jnp.bfloat16)
    adjb = _rb(adj_ref[...])
    T = [None] * _S_SUB
    for h in range(_K_STEP):
        if h > 0:
            Fb = [_rb(f) for f in F]
            newF = []
            for r in range(_S_SUB):
                acc = adjb[:, r * 5:r * 5 + 1] * Fb[0]
                for c in range(1, _S_SUB):
                    acc = acc + adjb[:, r * 5 + c:r * 5 + c + 1] * Fb[c]
                newF.append(acc)
            F = newF
        w16 = fhh16_scr[h]
        q = fhsq_scr[h]
        for c in range(_S_SUB):
            Mt = lax.dot_general(w16, F[c].astype(jnp.bfloat16),
                                 (((1,), (1,)), ((), ())),
                                 preferred_element_type=jnp.float32)
            fsqt = lax.dot_general(ones_row16, (F[c] * F[c]).astype(jnp.bfloat16),
                                   (((1,), (1,)), ((), ())),
                                   preferred_element_type=jnp.float32)
            e = jnp.exp(-(fsqt + q - 2.0 * Mt) / _D_IN / _TAO)
            T[c] = e if h == 0 else T[c] + e

    out = T[0][0:_N_FILTER, :]
    neg = jnp.float32(-1.0)
    blocked = [jnp.full((_N_FILTER, _BS), d == 0, jnp.bool_) for d in range(_S_SUB)]
    for i in range(1, _S_SUB):
        v = [jnp.where(blocked[d], neg, T[i][d * 8:(d + 1) * 8, :])
             for d in range(_S_SUB)]
        m = v[0]
        for d in range(1, _S_SUB):
            m = jnp.maximum(m, v[d])
        out = out + m
        found = jnp.zeros((_N_FILTER, _BS), jnp.bool_)
        for d in range(_S_SUB):
            hit = (v[d] == m) & jnp.logical_not(found)
            blocked[d] = blocked[d] | hit
            found = found | hit
    out_ref[...] = out


def _tc_compute(g3, adjs2d, valid, adjs_hidden, features_hidden, interpret=False):
    return pl.pallas_call(
        _tc_body,
        grid=(_GRID,),
        in_specs=[
            pl.BlockSpec((_S_SUB, _BS, _D_IN), lambda i: (0, i, 0)),
            pl.BlockSpec((_BS, _S_SUB * _S_SUB), lambda i: (i, 0)),
            pl.BlockSpec((_BS, _S_SUB), lambda i: (i, 0)),
            pl.BlockSpec((_N_FILTER, 10), lambda i: (0, 0)),
            pl.BlockSpec((_N_FILTER, _S_SUB, _D_IN), lambda i: (0, 0, 0)),
        ],
        out_specs=pl.BlockSpec((_N_FILTER, _BS), lambda i: (0, i)),
        out_shape=jax.ShapeDtypeStruct((_N_FILTER, _N_PAD), jnp.float32),
        scratch_shapes=[
            pltpu.VMEM((_K_STEP, 40, _D_IN), jnp.float32),
            pltpu.VMEM((_K_STEP, 40, _D_IN), jnp.bfloat16),
            pltpu.VMEM((_K_STEP, 40, 1), jnp.float32),
        ],
        compiler_params=pltpu.CompilerParams(
            dimension_semantics=("arbitrary",)),
        interpret=interpret,
    )(g3, adjs2d, valid, adjs_hidden, features_hidden)


def kernel(adjs, feature, idxs, adjs_hidden, features_hidden):
    pad = _N_PAD - _N_SUB
    idx32 = idxs.astype(jnp.int32)
    valid = jnp.pad((idx32 < _N_NODES).astype(jnp.float32),
                    ((0, pad), (0, 0)))
    idx_cm = jnp.pad(jnp.minimum(idx32, _N_NODES - 1).T,
                     ((0, 0), (0, pad))).reshape(-1)
    g_flat = _sc_gather(feature, idx_cm)
    g3 = g_flat.reshape(_S_SUB, _N_PAD, _D_IN)
    adjs2d = jnp.pad(adjs.reshape(_N_SUB, _S_SUB * _S_SUB),
                     ((0, pad), (0, 0)))
    out_t = _tc_compute(g3, adjs2d, valid, adjs_hidden, features_hidden)
    return out_t[:, :_N_SUB].T


# speedup vs baseline: 4.6100x; 1.1067x over previous
import functools

import jax
import jax.numpy as jnp
from jax import lax
from jax.experimental import pallas as pl
from jax.experimental.pallas import tpu as pltpu
from jax.experimental.pallas import tpu_sc as plsc

_N_FILTER = 8
_S_SUB = 5
_D_IN = 128
_K_STEP = 3
_TAO = 0.05
_N_NODES = 100000
_N_SUB = 50000

_N_PAD = 50176
_BS = 512
_GRID = _N_PAD // _BS

_N_SLICE = 2
_N_HALF = _N_PAD // _N_SLICE
_GRID_H = _N_HALF // _BS

_NW = 32
_N_FLAT = _S_SUB * _N_HALF
_ROWS_W = _N_FLAT // _NW
_CHUNK = 392
_NCH_W = _ROWS_W // _CHUNK

_PAIR = (
    (None, 0, 1, 2, 3),
    (0, None, 4, 5, 6),
    (1, 4, None, 7, 8),
    (2, 5, 7, None, 9),
    (3, 6, 8, 9, None),
)

def _rb(x):
    return x.astype(jnp.bfloat16).astype(jnp.float32)


def _sc_gather(table, idx_flat):
    mesh = plsc.VectorSubcoreMesh(core_axis_name="c", subcore_axis_name="s")

    @functools.partial(
        pl.kernel,
        mesh=mesh,
        out_type=jax.ShapeDtypeStruct((_N_FLAT, _D_IN), jnp.float32),
        scratch_types=[
            pltpu.VMEM((_CHUNK,), jnp.int32),
            pltpu.VMEM((_CHUNK,), jnp.int32),
            pltpu.VMEM((_CHUNK, _D_IN), jnp.float32),
            pltpu.VMEM((_CHUNK, _D_IN), jnp.float32),
            pltpu.SemaphoreType.DMA,
            pltpu.SemaphoreType.DMA,
            pltpu.SemaphoreType.DMA,
            pltpu.SemaphoreType.DMA,
        ],
    )
    def gk(table_hbm, idx_hbm, out_hbm, ixa, ixb, rwa, rwb, g0, g1, o0, o1):
        wid = lax.axis_index("s") * 2 + lax.axis_index("c")
        base = wid * _ROWS_W
        bufs = ((ixa, rwa, g0, o0), (ixb, rwb, g1, o1))

        for b in range(2):
            ix, rw, g, _o = bufs[b]
            pltpu.sync_copy(idx_hbm.at[pl.ds(base + b * _CHUNK, _CHUNK)], ix)
            pltpu.async_copy(table_hbm.at[ix], rw, g)

        def body(j, carry):
            for b in range(2):
                ix, rw, g, o = bufs[b]
                off = base + (2 * j + b) * _CHUNK
                pltpu.make_async_copy(table_hbm.at[ix], rw, g).wait()
                pltpu.async_copy(rw, out_hbm.at[pl.ds(off, _CHUNK)], o)

                @pl.when(j < _NCH_W // 2 - 1)
                def _():
                    pltpu.make_async_copy(
                        rw, out_hbm.at[pl.ds(off, _CHUNK)], o).wait()
                    pltpu.sync_copy(
                        idx_hbm.at[pl.ds(off + 2 * _CHUNK, _CHUNK)], ix)
                    pltpu.async_copy(table_hbm.at[ix], rw, g)
            return carry

        lax.fori_loop(0, _NCH_W // 2, body, 0)
        for b in range(2):
            ix, rw, g, o = bufs[b]
            pltpu.make_async_copy(
                rw,
                out_hbm.at[pl.ds(base + (_NCH_W - 2 + b) * _CHUNK, _CHUNK)],
                o).wait()

    return gk(table, idx_flat)


def _tc_body(g_ref, adj_ref, val_ref, ah_ref, fh_ref, out_ref, fhh_scr,
             fhh16_scr, fhsq_scr):
    @pl.when(pl.program_id(0) == 0)
    def _():
        sig = 1.0 / (1.0 + jnp.exp(-ah_ref[...]))
        sig = _rb(sig)
        for d in range(_S_SUB):
            for b in range(_N_FILTER):
                r = d * _N_FILTER + b
                fhh_scr[0, r:r + 1, :] = fh_ref[b, d:d + 1, :]
        for h in range(1, _K_STEP):
            for d in range(_S_SUB):
                acc = None
                for c in range(_S_SUB):
                    if c == d:
                        continue
                    k = _PAIR[d][c]
                    term = sig[:, k:k + 1] * _rb(fhh_scr[h - 1, c * 8:(c + 1) * 8, :])
                    acc = term if acc is None else acc + term
                fhh_scr[h, d * 8:(d + 1) * 8, :] = acc
        ones_col = jnp.ones((_D_IN, 1), jnp.bfloat16)
        for h in range(_K_STEP):
            w = fhh_scr[h]
            fhh16_scr[h, :, :] = w.astype(jnp.bfloat16)
            fhsq_scr[h, :, :] = lax.dot_general(
                (w * w).astype(jnp.bfloat16), ones_col, (((1,), (0,)), ((), ())),
                preferred_element_type=jnp.float32)

    F = [g_ref[c] * val_ref[:, c:c + 1] for c in range(_S_SUB)]

    ones_row16 = jnp.ones((1, _D_IN), jnp.bfloat16)
    adjb = _rb(adj_ref[...])
    T = [None] * _S_SUB
    for h in range(_K_STEP):
        if h > 0:
            Fb = [_rb(f) for f in F]
            newF = []
            for r in range(_S_SUB):
                acc = adjb[:, r * 5:r * 5 + 1] * Fb[0]
                for c in range(1, _S_SUB):
                    acc = acc + adjb[:, r * 5 + c:r * 5 + c + 1] * Fb[c]
                newF.append(acc)
            F = newF
        w16 = fhh16_scr[h]
        q = fhsq_scr[h]
        for c in range(_S_SUB):
            Mt = lax.dot_general(w16, F[c].astype(jnp.bfloat16),
                                 (((1,), (1,)), ((), ())),
                                 preferred_element_type=jnp.float32)
            fsqt = lax.dot_general(ones_row16, (F[c] * F[c]).astype(jnp.bfloat16),
                                   (((1,), (1,)), ((), ())),
                                   preferred_element_type=jnp.float32)
            e = jnp.exp(-(fsqt + q - 2.0 * Mt) / _D_IN / _TAO)
            T[c] = e if h == 0 else T[c] + e

    out = T[0][0:_N_FILTER, :]
    neg = jnp.float32(-1.0)
    blocked = [jnp.full((_N_FILTER, _BS), d == 0, jnp.bool_) for d in range(_S_SUB)]
    for i in range(1, _S_SUB):
        v = [jnp.where(blocked[d], neg, T[i][d * 8:(d + 1) * 8, :])
             for d in range(_S_SUB)]
        m = v[0]
        for d in range(1, _S_SUB):
            m = jnp.maximum(m, v[d])
        out = out + m
        found = jnp.zeros((_N_FILTER, _BS), jnp.bool_)
        for d in range(_S_SUB):
            hit = (v[d] == m) & jnp.logical_not(found)
            blocked[d] = blocked[d] | hit
            found = found | hit
    out_ref[...] = out


def _tc_compute(g3, adjs2d, valid, adjs_hidden, features_hidden, interpret=False):
    n = g3.shape[1]
    return pl.pallas_call(
        _tc_body,
        grid=(n // _BS,),
        in_specs=[
            pl.BlockSpec((_S_SUB, _BS, _D_IN), lambda i: (0, i, 0)),
            pl.BlockSpec((_BS, _S_SUB * _S_SUB), lambda i: (i, 0)),
            pl.BlockSpec((_BS, _S_SUB), lambda i: (i, 0)),
            pl.BlockSpec((_N_FILTER, 10), lambda i: (0, 0)),
            pl.BlockSpec((_N_FILTER, _S_SUB, _D_IN), lambda i: (0, 0, 0)),
        ],
        out_specs=pl.BlockSpec((_N_FILTER, _BS), lambda i: (0, i)),
        out_shape=jax.ShapeDtypeStruct((_N_FILTER, n), jnp.float32),
        scratch_shapes=[
            pltpu.VMEM((_K_STEP, 40, _D_IN), jnp.float32),
            pltpu.VMEM((_K_STEP, 40, _D_IN), jnp.bfloat16),
            pltpu.VMEM((_K_STEP, 40, 1), jnp.float32),
        ],
        compiler_params=pltpu.CompilerParams(
            dimension_semantics=("arbitrary",)),
        interpret=interpret,
    )(g3, adjs2d, valid, adjs_hidden, features_hidden)


def kernel(adjs, feature, idxs, adjs_hidden, features_hidden):
    pad = _N_PAD - _N_SUB
    idx32 = idxs.astype(jnp.int32)
    valid = jnp.pad((idx32 < _N_NODES).astype(jnp.float32),
                    ((0, pad), (0, 0)))
    idx_t = jnp.pad(jnp.minimum(idx32, _N_NODES - 1).T,
                    ((0, 0), (0, pad)))
    adjs2d = jnp.pad(adjs.reshape(_N_SUB, _S_SUB * _S_SUB),
                     ((0, pad), (0, 0)))
    outs = []
    for k in range(_N_SLICE):
        lo = k * _N_HALF
        idx_cm = idx_t[:, lo:lo + _N_HALF].reshape(-1)
        g_flat = _sc_gather(feature, idx_cm)
        g3 = g_flat.reshape(_S_SUB, _N_HALF, _D_IN)
        outs.append(_tc_compute(g3, adjs2d[lo:lo + _N_HALF],
                                valid[lo:lo + _N_HALF],
                                adjs_hidden, features_hidden))
    out_t = jnp.concatenate(outs, axis=1)
    return out_t[:, :_N_SUB].T
